# Initial kernel scaffold; baseline (speedup 1.0000x reference)
import jax
import jax.numpy as jnp
from jax.experimental import pallas as pl


def kernel(inputs, edge_index, batch, edge_weight, W0, b0, gnw0, gnb0, gna0, W1, b1, gnw1, gnb1, gna1, W2, b2, gnw2, gnb2, gna2, Wdin, bdin, Wd1, bd1, Wdout, bdout):
    raise NotImplementedError("write your pallas kernel here")


# speedup vs baseline: 4.3415x; 4.3415x over previous
import functools

import jax
import jax.numpy as jnp
from jax import lax
from jax.experimental import pallas as pl
from jax.experimental.pallas import tpu as pltpu
from jax.experimental.pallas import tpu_sc as plsc

NN = 10000
EE = 160000
DD = 256
GG = 16
CC = 16
HALF = 128

BS = 1000
NBLK = NN // BS

EPAD = 163840
EPT_D = EPAD // 32
NCH_D = EPT_D // 128
EPT_A = EPAD // 16
NCH_A = EPT_A // 128
NPAD = 10240
RPT = NPAD // 16
ZR_D = 10240 // 16


def _onehot(batch_blk):
    return (batch_blk == lax.broadcasted_iota(jnp.int32, (1, GG), 1)).astype(
        jnp.float32)


def _segsum(p, v):
    return lax.dot_general(p, v, (((0,), (0,)), ((), ())),
                           precision=lax.Precision.HIGHEST,
                           preferred_element_type=jnp.float32)


@functools.cache
def _get_sc_deg():
    mesh = plsc.VectorSubcoreMesh(core_axis_name="c", subcore_axis_name="s")
    return pl.kernel(
        _sc_deg_body,
        out_type=jax.ShapeDtypeStruct((2, 10240), jnp.float32),
        mesh=mesh,
        scratch_types=[
            pltpu.VMEM((128,), jnp.int32),
            pltpu.VMEM((128,), jnp.float32),
            pltpu.VMEM_SHARED((10240,), jnp.float32),
        ],
    )


def _sc_deg_body(dst_hbm, w_hbm, z_hbm, deg_hbm, dbuf, wbuf, deg_s):
    c = lax.axis_index("c")
    s = lax.axis_index("s")
    pltpu.sync_copy(z_hbm, deg_s.at[pl.ds(s * ZR_D, ZR_D)])
    plsc.subcore_barrier()
    base = (c * 16 + s) * EPT_D

    def chunk(g, carry):
        off = base + g * 128
        pltpu.sync_copy(dst_hbm.at[pl.ds(off, 128)], dbuf)
        pltpu.sync_copy(w_hbm.at[pl.ds(off, 128)], wbuf)
        pltpu.sync_copy(wbuf, deg_s.at[dbuf], add=True)
        return carry

    lax.fori_loop(0, NCH_D, chunk, 0)
    plsc.subcore_barrier()
    pltpu.sync_copy(deg_s.at[pl.ds(s * ZR_D, ZR_D)],
                    deg_hbm.at[c, pl.ds(s * ZR_D, ZR_D)])


@functools.cache
def _get_sc_agg():
    mesh = plsc.VectorSubcoreMesh(core_axis_name="c", subcore_axis_name="s")
    return pl.kernel(
        _sc_agg_body,
        out_type=jax.ShapeDtypeStruct((2, NPAD, HALF), jnp.float32),
        mesh=mesh,
        scratch_types=[
            pltpu.VMEM((128,), jnp.int32),
            pltpu.VMEM((128,), jnp.int32),
            pltpu.VMEM((128,), jnp.float32),
            pltpu.VMEM((128, HALF), jnp.float32),
            pltpu.VMEM_SHARED((NPAD, HALF), jnp.float32),
            pltpu.SemaphoreType.DMA,
        ],
    )


def _sc_agg_body(y_hbm, src_hbm, dst_hbm, w_hbm, zr_hbm, acc_hbm,
                 sbuf, dbuf, wbuf, rows, acc_s, sem):
    c = lax.axis_index("c")
    s = lax.axis_index("s")
    pltpu.sync_copy(zr_hbm, acc_s.at[pl.ds(s * RPT, RPT)])
    plsc.subcore_barrier()
    base = s * EPT_A
    coff = c * NN

    def chunk(g, carry):
        off = base + g * 128
        pltpu.sync_copy(src_hbm.at[pl.ds(off, 128)], sbuf)
        pltpu.sync_copy(dst_hbm.at[pl.ds(off, 128)], dbuf)
        pltpu.sync_copy(w_hbm.at[pl.ds(off, 128)], wbuf)
        for j in range(8):
            sl = pl.ds(j * 16, 16)
            sbuf[sl] = sbuf[sl] + coff
        pltpu.async_copy(y_hbm.at[sbuf], rows, sem).wait()

        def mul(m, carry2):
            wv = wbuf[pl.ds(m * 16, 16)]
            for l in range(16):
                wk = wv[l]
                row = m * 16 + l
                for j in range(8):
                    sl = pl.ds(j * 16, 16)
                    rows[row, sl] = rows[row, sl] * wk
            return carry2

        lax.fori_loop(0, 8, mul, 0)
        pltpu.sync_copy(rows, acc_s.at[dbuf], add=True)
        return carry

    lax.fori_loop(0, NCH_A, chunk, 0)
    plsc.subcore_barrier()
    pltpu.sync_copy(acc_s.at[pl.ds(s * RPT, RPT)],
                    acc_hbm.at[c, pl.ds(s * RPT, RPT)])


def _kdis_body(degp_ref, dis_ref):
    a = degp_ref[...]
    tot = a[0:1, :] + a[1:2, :] + 1.0
    dis_ref[...] = lax.rsqrt(tot)


def _kdis(degp):
    return pl.pallas_call(
        _kdis_body,
        out_shape=jax.ShapeDtypeStruct((1, 10240), jnp.float32),
    )(degp)


def _k1_body(x_ref, w_ref, dis_ref, y_ref):
    xw = jnp.dot(x_ref[...], w_ref[...], preferred_element_type=jnp.float32)
    y = xw * dis_ref[...]
    y_ref[0] = y[:, :HALF]
    y_ref[1] = y[:, HALF:]


def _k1(x, W, dis2):
    return pl.pallas_call(
        _k1_body,
        grid=(NBLK,),
        in_specs=[pl.BlockSpec((BS, DD), lambda i: (i, 0)),
                  pl.BlockSpec((DD, DD), lambda i: (0, 0)),
                  pl.BlockSpec((BS, 1), lambda i: (i, 0))],
        out_specs=pl.BlockSpec((2, BS, HALF), lambda i: (0, i, 0)),
        out_shape=jax.ShapeDtypeStruct((2, NN, HALF), jnp.float32),
    )(x, W, dis2)


def _k2a_body(acc_ref, y_ref, dis_ref, b_ref, bat_ref, hpre_ref, s1_ref):
    i = pl.program_id(0)
    acc = jnp.concatenate([acc_ref[0], acc_ref[1]], axis=1)
    y = jnp.concatenate([y_ref[0], y_ref[1]], axis=1)
    hpre = dis_ref[...] * (acc + y) + b_ref[...]
    hpre_ref[...] = hpre
    p = _onehot(bat_ref[...])

    @pl.when(i == 0)
    def _():
        s1_ref[...] = jnp.zeros_like(s1_ref)

    s1_ref[...] += _segsum(p, hpre)


def _k2a(acc, y2, dis2, b, batch2):
    return pl.pallas_call(
        _k2a_body,
        grid=(NBLK,),
        in_specs=[pl.BlockSpec((2, BS, HALF), lambda i: (0, i, 0)),
                  pl.BlockSpec((2, BS, HALF), lambda i: (0, i, 0)),
                  pl.BlockSpec((BS, 1), lambda i: (i, 0)),
                  pl.BlockSpec((1, DD), lambda i: (0, 0)),
                  pl.BlockSpec((BS, 1), lambda i: (i, 0))],
        out_specs=[pl.BlockSpec((BS, DD), lambda i: (i, 0)),
                   pl.BlockSpec((GG, DD), lambda i: (0, 0))],
        out_shape=[jax.ShapeDtypeStruct((NN, DD), jnp.float32),
                   jax.ShapeDtypeStruct((GG, DD), jnp.float32)],
    )(acc, y2, dis2, b, batch2)


def _k2b_body(hpre_ref, s1_ref, cnt_ref, ga_ref, bat_ref, out0_ref, s2_ref):
    i = pl.program_id(0)
    mean = s1_ref[...] / cnt_ref[...]
    m2 = mean * ga_ref[...]
    p = _onehot(bat_ref[...])
    out0 = hpre_ref[...] - jnp.dot(p, m2, precision=lax.Precision.HIGHEST,
                                   preferred_element_type=jnp.float32)
    out0_ref[...] = out0

    @pl.when(i == 0)
    def _():
        s2_ref[...] = jnp.zeros_like(s2_ref)

    s2_ref[...] += _segsum(p, out0 * out0)


def _k2b(hpre, S1, cnt, ga, batch2):
    return pl.pallas_call(
        _k2b_body,
        grid=(NBLK,),
        in_specs=[pl.BlockSpec((BS, DD), lambda i: (i, 0)),
                  pl.BlockSpec((GG, DD), lambda i: (0, 0)),
                  pl.BlockSpec((GG, 1), lambda i: (0, 0)),
                  pl.BlockSpec((1, DD), lambda i: (0, 0)),
                  pl.BlockSpec((BS, 1), lambda i: (i, 0))],
        out_specs=[pl.BlockSpec((BS, DD), lambda i: (i, 0)),
                   pl.BlockSpec((GG, DD), lambda i: (0, 0))],
        out_shape=[jax.ShapeDtypeStruct((NN, DD), jnp.float32),
                   jax.ShapeDtypeStruct((GG, DD), jnp.float32)],
    )(hpre, S1, cnt, ga, batch2)


def _k2c_body(out0_ref, s2_ref, cnt_ref, gw_ref, gb_ref, bat_ref, xres_ref,
              h_ref, *, res):
    inv = lax.rsqrt(s2_ref[...] / cnt_ref[...] + 1e-5)
    p = _onehot(bat_ref[...])
    fb = jnp.dot(p, gw_ref[...] * inv, precision=lax.Precision.HIGHEST,
                 preferred_element_type=jnp.float32)
    h = jnp.maximum(out0_ref[...] * fb + gb_ref[...], 0.0)
    if res:
        h = h + xres_ref[...]
    h_ref[...] = h


def _k2c(out0, S2, cnt, gw, gb, batch2, xres, res):
    return pl.pallas_call(
        functools.partial(_k2c_body, res=res),
        grid=(NBLK,),
        in_specs=[pl.BlockSpec((BS, DD), lambda i: (i, 0)),
                  pl.BlockSpec((GG, DD), lambda i: (0, 0)),
                  pl.BlockSpec((GG, 1), lambda i: (0, 0)),
                  pl.BlockSpec((1, DD), lambda i: (0, 0)),
                  pl.BlockSpec((1, DD), lambda i: (0, 0)),
                  pl.BlockSpec((BS, 1), lambda i: (i, 0)),
                  pl.BlockSpec((BS, DD), lambda i: (i, 0))],
        out_specs=pl.BlockSpec((BS, DD), lambda i: (i, 0)),
        out_shape=jax.ShapeDtypeStruct((NN, DD), jnp.float32),
    )(out0, S2, cnt, gw, gb, batch2, xres)


def _kcnt_body(bat_ref, cnt_ref):
    i = pl.program_id(0)
    p = _onehot(bat_ref[...])

    @pl.when(i == 0)
    def _():
        cnt_ref[...] = jnp.zeros_like(cnt_ref)

    cnt_ref[...] += _segsum(p, jnp.ones((BS, HALF), jnp.float32))


def _kcnt(batch2):
    return pl.pallas_call(
        _kcnt_body,
        grid=(NBLK,),
        in_specs=[pl.BlockSpec((BS, 1), lambda i: (i, 0))],
        out_specs=pl.BlockSpec((GG, HALF), lambda i: (0, 0)),
        out_shape=jax.ShapeDtypeStruct((GG, HALF), jnp.float32),
    )(batch2)


def _k3a_body(h_ref, bat_ref, s_ref):
    i = pl.program_id(0)
    p = _onehot(bat_ref[...])

    @pl.when(i == 0)
    def _():
        s_ref[...] = jnp.zeros_like(s_ref)

    s_ref[...] += _segsum(p, h_ref[...])


def _k3a(h, batch2):
    return pl.pallas_call(
        _k3a_body,
        grid=(NBLK,),
        in_specs=[pl.BlockSpec((BS, DD), lambda i: (i, 0)),
                  pl.BlockSpec((BS, 1), lambda i: (i, 0))],
        out_specs=pl.BlockSpec((GG, DD), lambda i: (0, 0)),
        out_shape=jax.ShapeDtypeStruct((GG, DD), jnp.float32),
    )(h, batch2)


def _k3b_body(pool_ref, cnt_ref, wdin_ref, bdin_ref, wd1_ref, bd1_ref,
              wdout_ref, bdout_ref, o_ref):
    m = pool_ref[...] / cnt_ref[...]
    z = jnp.maximum(
        jnp.dot(m, wdin_ref[...], preferred_element_type=jnp.float32)
        + bdin_ref[...], 0.0)
    z = jnp.maximum(
        jnp.dot(z, wd1_ref[...], preferred_element_type=jnp.float32)
        + bd1_ref[...], 0.0)
    o_ref[...] = (jnp.dot(z, wdout_ref[...], preferred_element_type=jnp.float32)
                  + bdout_ref[...])


def _k3b(pool, cnt, Wdin, bdin, Wd1, bd1, Wdout, bdout):
    return pl.pallas_call(
        _k3b_body,
        out_shape=jax.ShapeDtypeStruct((GG, CC), jnp.float32),
    )(pool, cnt, Wdin, bdin, Wd1, bd1, Wdout, bdout)


def kernel(inputs, edge_index, batch, edge_weight, W0, b0, gnw0, gnb0, gna0,
           W1, b1, gnw1, gnb1, gna1, W2, b2, gnw2, gnb2, gna2,
           Wdin, bdin, Wd1, bd1, Wdout, bdout):
    pad = EPAD - EE
    srcp = jnp.concatenate([edge_index[0], jnp.zeros((pad,), jnp.int32)])
    dstp = jnp.concatenate([edge_index[1], jnp.zeros((pad,), jnp.int32)])
    wp = jnp.concatenate([edge_weight, jnp.zeros((pad,), jnp.float32)])
    zdeg = jnp.zeros((ZR_D,), jnp.float32)
    zrows = jnp.zeros((RPT, HALF), jnp.float32)
    batch2 = batch.reshape(NN, 1)

    degp = _get_sc_deg()(dstp, wp, zdeg)
    dis2 = _kdis(degp).reshape(10240, 1)[:NN]
    cnt = _kcnt(batch2)[:, :1]

    h = inputs
    layers = [(W0, b0, gnw0, gnb0, gna0, False),
              (W1, b1, gnw1, gnb1, gna1, True),
              (W2, b2, gnw2, gnb2, gna2, True)]
    for (W, b, gw, gb, ga, res) in layers:
        y2 = _k1(h, W, dis2)
        acc = _get_sc_agg()(y2.reshape(2 * NN, HALF), srcp, dstp, wp, zrows)
        hpre, S1 = _k2a(acc, y2, dis2, b.reshape(1, DD), batch2)
        out0, S2 = _k2b(hpre, S1, cnt, ga.reshape(1, DD), batch2)
        h = _k2c(out0, S2, cnt, gw.reshape(1, DD), gb.reshape(1, DD),
                 batch2, h, res)

    pool = _k3a(h, batch2)
    return _k3b(pool, cnt, Wdin, bdin.reshape(1, DD), Wd1, bd1.reshape(1, DD),
                Wdout, bdout.reshape(1, CC))


# speedup vs baseline: 7.9310x; 1.8268x over previous
import functools

import jax
import jax.numpy as jnp
from jax import lax
from jax.experimental import pallas as pl
from jax.experimental.pallas import tpu as pltpu
from jax.experimental.pallas import tpu_sc as plsc

NN = 10000
EE = 160000
DD = 256
GG = 16
CC = 16
HALF = 128

BS = 1000
NBLK = NN // BS

EPAD = 163840
EPT_D = EPAD // 32
NCH_D = EPT_D // 128
EPT_A = EPAD // 16
NCH_A = EPT_A // 128
NPAD = 10240
RPT = NPAD // 16
ZR_D = 10240 // 16


def _onehot(batch_blk):
    return (batch_blk == lax.broadcasted_iota(jnp.int32, (1, GG), 1)).astype(
        jnp.float32)


def _segsum(p, v):
    return lax.dot_general(p, v, (((0,), (0,)), ((), ())),
                           precision=lax.Precision.HIGHEST,
                           preferred_element_type=jnp.float32)


@functools.cache
def _get_sc_deg():
    mesh = plsc.VectorSubcoreMesh(core_axis_name="c", subcore_axis_name="s")
    return pl.kernel(
        _sc_deg_body,
        out_type=jax.ShapeDtypeStruct((2, 10240), jnp.float32),
        mesh=mesh,
        scratch_types=[
            pltpu.VMEM((128,), jnp.int32),
            pltpu.VMEM((128,), jnp.float32),
            pltpu.VMEM_SHARED((10240,), jnp.float32),
        ],
    )


def _sc_deg_body(dst_hbm, w_hbm, z_hbm, deg_hbm, dbuf, wbuf, deg_s):
    c = lax.axis_index("c")
    s = lax.axis_index("s")
    pltpu.sync_copy(z_hbm, deg_s.at[pl.ds(s * ZR_D, ZR_D)])
    plsc.subcore_barrier()
    base = (c * 16 + s) * EPT_D

    def chunk(g, carry):
        off = base + g * 128
        pltpu.sync_copy(dst_hbm.at[pl.ds(off, 128)], dbuf)
        pltpu.sync_copy(w_hbm.at[pl.ds(off, 128)], wbuf)
        pltpu.sync_copy(wbuf, deg_s.at[dbuf], add=True)
        return carry

    lax.fori_loop(0, NCH_D, chunk, 0)
    plsc.subcore_barrier()
    pltpu.sync_copy(deg_s.at[pl.ds(s * ZR_D, ZR_D)],
                    deg_hbm.at[c, pl.ds(s * ZR_D, ZR_D)])


CHPT = NCH_A


@functools.cache
def _get_sc_agg():
    mesh = plsc.VectorSubcoreMesh(core_axis_name="c", subcore_axis_name="s")
    return pl.kernel(
        _sc_agg_body,
        out_type=jax.ShapeDtypeStruct((2, NPAD, HALF), jnp.float32),
        mesh=mesh,
        scratch_types=[
            pltpu.VMEM((128,), jnp.int32),
            pltpu.VMEM((128,), jnp.int32),
            pltpu.VMEM((128,), jnp.int32),
            pltpu.VMEM((128,), jnp.int32),
            pltpu.VMEM((128,), jnp.float32),
            pltpu.VMEM((128,), jnp.float32),
            pltpu.VMEM((128, HALF), jnp.float32),
            pltpu.VMEM((128, HALF), jnp.float32),
            pltpu.VMEM_SHARED((NPAD, HALF), jnp.float32),
            pltpu.SemaphoreType.DMA,
            pltpu.SemaphoreType.DMA,
            pltpu.SemaphoreType.DMA,
            pltpu.SemaphoreType.DMA,
            pltpu.SemaphoreType.DMA,
            pltpu.SemaphoreType.DMA,
        ],
    )


def _sc_agg_body(y_hbm, src2_hbm, dst_hbm, w_hbm, zr_hbm, acc_hbm,
                 sbuf0, sbuf1, dbuf0, dbuf1, wbuf0, wbuf1, rows0, rows1,
                 acc_s, sg0, sg1, si0, si1, sd0, sd1):
    c = lax.axis_index("c")
    s = lax.axis_index("s")
    sbuf = (sbuf0, sbuf1)
    dbuf = (dbuf0, dbuf1)
    wbuf = (wbuf0, wbuf1)
    rows = (rows0, rows1)
    sg = (sg0, sg1)
    si = (si0, si1)
    sd = (sd0, sd1)
    base = s * CHPT

    def src_row(g):
        return src2_hbm.at[c, base + g]

    def dst_row(g):
        return dst_hbm.at[base + g]

    def w_row(g):
        return w_hbm.at[base + g]

    pltpu.sync_copy(src_row(0), sbuf0)
    pltpu.async_copy(y_hbm.at[sbuf0], rows0, sg0)
    pltpu.async_copy(dst_row(0), dbuf0, sd0)
    pltpu.async_copy(w_row(0), wbuf0, sd0)
    pltpu.async_copy(src_row(1), sbuf1, si1)
    pltpu.async_copy(dst_row(1), dbuf1, sd1)
    pltpu.async_copy(w_row(1), wbuf1, sd1)
    pltpu.sync_copy(zr_hbm, acc_s.at[pl.ds(s * RPT, RPT)])
    plsc.subcore_barrier()

    def pair(i, carry):
        g0 = i * 2
        for par in range(2):
            g = g0 + par
            q = 1 - par

            @pl.when(g + 1 < NCH_A)
            def _(g=g, par=par, q=q):
                pltpu.make_async_copy(src_row(g + 1), sbuf[q], si[q]).wait()
                pltpu.async_copy(y_hbm.at[sbuf[q]], rows[q], sg[q])

            pltpu.make_async_copy(y_hbm.at[sbuf[par]], rows[par],
                                  sg[par]).wait()

            @pl.when(g + 2 < NCH_A)
            def _(g=g, par=par):
                pltpu.async_copy(src_row(g + 2), sbuf[par], si[par])

            pltpu.make_async_copy(dst_row(g), dbuf[par], sd[par]).wait()
            pltpu.make_async_copy(w_row(g), wbuf[par], sd[par]).wait()

            def mul(m, carry2, par=par):
                wv = wbuf[par][pl.ds(m * 16, 16)]
                for l in range(16):
                    wk = wv[l]
                    row = m * 16 + l
                    for j in range(8):
                        sl = pl.ds(j * 16, 16)
                        rows[par][row, sl] = rows[par][row, sl] * wk
                return carry2

            lax.fori_loop(0, 8, mul, 0)
            pltpu.sync_copy(rows[par], acc_s.at[dbuf[par]], add=True)

            @pl.when(g + 2 < NCH_A)
            def _(g=g, par=par):
                pltpu.async_copy(dst_row(g + 2), dbuf[par], sd[par])
                pltpu.async_copy(w_row(g + 2), wbuf[par], sd[par])
        return carry

    lax.fori_loop(0, NCH_A // 2, pair, 0)
    plsc.subcore_barrier()
    pltpu.sync_copy(acc_s.at[pl.ds(s * RPT, RPT)],
                    acc_hbm.at[c, pl.ds(s * RPT, RPT)])


def _kdis_body(degp_ref, dis_ref):
    a = degp_ref[...]
    tot = a[0:1, :] + a[1:2, :] + 1.0
    dis_ref[...] = lax.rsqrt(tot)


def _kdis(degp):
    return pl.pallas_call(
        _kdis_body,
        out_shape=jax.ShapeDtypeStruct((1, 10240), jnp.float32),
    )(degp)


def _k1_body(x_ref, w_ref, dis_ref, y_ref):
    xw = jnp.dot(x_ref[...], w_ref[...], preferred_element_type=jnp.float32)
    y = xw * dis_ref[...]
    y_ref[0] = y[:, :HALF]
    y_ref[1] = y[:, HALF:]


def _k1(x, W, dis2):
    return pl.pallas_call(
        _k1_body,
        grid=(NBLK,),
        in_specs=[pl.BlockSpec((BS, DD), lambda i: (i, 0)),
                  pl.BlockSpec((DD, DD), lambda i: (0, 0)),
                  pl.BlockSpec((BS, 1), lambda i: (i, 0))],
        out_specs=pl.BlockSpec((2, BS, HALF), lambda i: (0, i, 0)),
        out_shape=jax.ShapeDtypeStruct((2, NN, HALF), jnp.float32),
    )(x, W, dis2)


def _k2a_body(acc_ref, y_ref, dis_ref, b_ref, bat_ref, hpre_ref, s1_ref):
    i = pl.program_id(0)
    acc = jnp.concatenate([acc_ref[0], acc_ref[1]], axis=1)
    y = jnp.concatenate([y_ref[0], y_ref[1]], axis=1)
    hpre = dis_ref[...] * (acc + y) + b_ref[...]
    hpre_ref[...] = hpre
    p = _onehot(bat_ref[...])

    @pl.when(i == 0)
    def _():
        s1_ref[...] = jnp.zeros_like(s1_ref)

    s1_ref[...] += _segsum(p, hpre)


def _k2a(acc, y2, dis2, b, batch2):
    return pl.pallas_call(
        _k2a_body,
        grid=(NBLK,),
        in_specs=[pl.BlockSpec((2, BS, HALF), lambda i: (0, i, 0)),
                  pl.BlockSpec((2, BS, HALF), lambda i: (0, i, 0)),
                  pl.BlockSpec((BS, 1), lambda i: (i, 0)),
                  pl.BlockSpec((1, DD), lambda i: (0, 0)),
                  pl.BlockSpec((BS, 1), lambda i: (i, 0))],
        out_specs=[pl.BlockSpec((BS, DD), lambda i: (i, 0)),
                   pl.BlockSpec((GG, DD), lambda i: (0, 0))],
        out_shape=[jax.ShapeDtypeStruct((NN, DD), jnp.float32),
                   jax.ShapeDtypeStruct((GG, DD), jnp.float32)],
    )(acc, y2, dis2, b, batch2)


def _k2b_body(hpre_ref, s1_ref, cnt_ref, ga_ref, bat_ref, out0_ref, s2_ref):
    i = pl.program_id(0)
    mean = s1_ref[...] / cnt_ref[...]
    m2 = mean * ga_ref[...]
    p = _onehot(bat_ref[...])
    out0 = hpre_ref[...] - jnp.dot(p, m2, precision=lax.Precision.HIGHEST,
                                   preferred_element_type=jnp.float32)
    out0_ref[...] = out0

    @pl.when(i == 0)
    def _():
        s2_ref[...] = jnp.zeros_like(s2_ref)

    s2_ref[...] += _segsum(p, out0 * out0)


def _k2b(hpre, S1, cnt, ga, batch2):
    return pl.pallas_call(
        _k2b_body,
        grid=(NBLK,),
        in_specs=[pl.BlockSpec((BS, DD), lambda i: (i, 0)),
                  pl.BlockSpec((GG, DD), lambda i: (0, 0)),
                  pl.BlockSpec((GG, 1), lambda i: (0, 0)),
                  pl.BlockSpec((1, DD), lambda i: (0, 0)),
                  pl.BlockSpec((BS, 1), lambda i: (i, 0))],
        out_specs=[pl.BlockSpec((BS, DD), lambda i: (i, 0)),
                   pl.BlockSpec((GG, DD), lambda i: (0, 0))],
        out_shape=[jax.ShapeDtypeStruct((NN, DD), jnp.float32),
                   jax.ShapeDtypeStruct((GG, DD), jnp.float32)],
    )(hpre, S1, cnt, ga, batch2)


def _k2c_body(out0_ref, s2_ref, cnt_ref, gw_ref, gb_ref, bat_ref, xres_ref,
              h_ref, *, res):
    inv = lax.rsqrt(s2_ref[...] / cnt_ref[...] + 1e-5)
    p = _onehot(bat_ref[...])
    fb = jnp.dot(p, gw_ref[...] * inv, precision=lax.Precision.HIGHEST,
                 preferred_element_type=jnp.float32)
    h = jnp.maximum(out0_ref[...] * fb + gb_ref[...], 0.0)
    if res:
        h = h + xres_ref[...]
    h_ref[...] = h


def _k2c(out0, S2, cnt, gw, gb, batch2, xres, res):
    return pl.pallas_call(
        functools.partial(_k2c_body, res=res),
        grid=(NBLK,),
        in_specs=[pl.BlockSpec((BS, DD), lambda i: (i, 0)),
                  pl.BlockSpec((GG, DD), lambda i: (0, 0)),
                  pl.BlockSpec((GG, 1), lambda i: (0, 0)),
                  pl.BlockSpec((1, DD), lambda i: (0, 0)),
                  pl.BlockSpec((1, DD), lambda i: (0, 0)),
                  pl.BlockSpec((BS, 1), lambda i: (i, 0)),
                  pl.BlockSpec((BS, DD), lambda i: (i, 0))],
        out_specs=pl.BlockSpec((BS, DD), lambda i: (i, 0)),
        out_shape=jax.ShapeDtypeStruct((NN, DD), jnp.float32),
    )(out0, S2, cnt, gw, gb, batch2, xres)


def _kcnt_body(bat_ref, cnt_ref):
    i = pl.program_id(0)
    p = _onehot(bat_ref[...])

    @pl.when(i == 0)
    def _():
        cnt_ref[...] = jnp.zeros_like(cnt_ref)

    cnt_ref[...] += _segsum(p, jnp.ones((BS, HALF), jnp.float32))


def _kcnt(batch2):
    return pl.pallas_call(
        _kcnt_body,
        grid=(NBLK,),
        in_specs=[pl.BlockSpec((BS, 1), lambda i: (i, 0))],
        out_specs=pl.BlockSpec((GG, HALF), lambda i: (0, 0)),
        out_shape=jax.ShapeDtypeStruct((GG, HALF), jnp.float32),
    )(batch2)


def _k3a_body(h_ref, bat_ref, s_ref):
    i = pl.program_id(0)
    p = _onehot(bat_ref[...])

    @pl.when(i == 0)
    def _():
        s_ref[...] = jnp.zeros_like(s_ref)

    s_ref[...] += _segsum(p, h_ref[...])


def _k3a(h, batch2):
    return pl.pallas_call(
        _k3a_body,
        grid=(NBLK,),
        in_specs=[pl.BlockSpec((BS, DD), lambda i: (i, 0)),
                  pl.BlockSpec((BS, 1), lambda i: (i, 0))],
        out_specs=pl.BlockSpec((GG, DD), lambda i: (0, 0)),
        out_shape=jax.ShapeDtypeStruct((GG, DD), jnp.float32),
    )(h, batch2)


def _k3b_body(pool_ref, cnt_ref, wdin_ref, bdin_ref, wd1_ref, bd1_ref,
              wdout_ref, bdout_ref, o_ref):
    m = pool_ref[...] / cnt_ref[...]
    z = jnp.maximum(
        jnp.dot(m, wdin_ref[...], preferred_element_type=jnp.float32)
        + bdin_ref[...], 0.0)
    z = jnp.maximum(
        jnp.dot(z, wd1_ref[...], preferred_element_type=jnp.float32)
        + bd1_ref[...], 0.0)
    o_ref[...] = (jnp.dot(z, wdout_ref[...], preferred_element_type=jnp.float32)
                  + bdout_ref[...])


def _k3b(pool, cnt, Wdin, bdin, Wd1, bd1, Wdout, bdout):
    return pl.pallas_call(
        _k3b_body,
        out_shape=jax.ShapeDtypeStruct((GG, CC), jnp.float32),
    )(pool, cnt, Wdin, bdin, Wd1, bd1, Wdout, bdout)


def kernel(inputs, edge_index, batch, edge_weight, W0, b0, gnw0, gnb0, gna0,
           W1, b1, gnw1, gnb1, gna1, W2, b2, gnw2, gnb2, gna2,
           Wdin, bdin, Wd1, bd1, Wdout, bdout):
    pad = EPAD - EE
    srcp = jnp.concatenate([edge_index[0], jnp.zeros((pad,), jnp.int32)])
    dstp = jnp.concatenate([edge_index[1], jnp.zeros((pad,), jnp.int32)])
    wp = jnp.concatenate([edge_weight, jnp.zeros((pad,), jnp.float32)])
    zdeg = jnp.zeros((ZR_D,), jnp.float32)
    zrows = jnp.zeros((RPT, HALF), jnp.float32)
    batch2 = batch.reshape(NN, 1)

    src2 = jnp.stack([srcp, srcp + NN]).reshape(2, EPAD // 128, 128)
    dstr = dstp.reshape(EPAD // 128, 128)
    wr = wp.reshape(EPAD // 128, 128)

    degp = _get_sc_deg()(dstp, wp, zdeg)
    dis2 = _kdis(degp).reshape(10240, 1)[:NN]
    cnt = _kcnt(batch2)[:, :1]

    h = inputs
    layers = [(W0, b0, gnw0, gnb0, gna0, False),
              (W1, b1, gnw1, gnb1, gna1, True),
              (W2, b2, gnw2, gnb2, gna2, True)]
    for (W, b, gw, gb, ga, res) in layers:
        y2 = _k1(h, W, dis2)
        acc = _get_sc_agg()(y2.reshape(2 * NN, HALF), src2, dstr, wr, zrows)
        hpre, S1 = _k2a(acc, y2, dis2, b.reshape(1, DD), batch2)
        out0, S2 = _k2b(hpre, S1, cnt, ga.reshape(1, DD), batch2)
        h = _k2c(out0, S2, cnt, gw.reshape(1, DD), gb.reshape(1, DD),
                 batch2, h, res)

    pool = _k3a(h, batch2)
    return _k3b(pool, cnt, Wdin, bdin.reshape(1, DD), Wd1, bd1.reshape(1, DD),
                Wdout, bdout.reshape(1, CC))


# speedup vs baseline: 7.9378x; 1.0009x over previous
import functools

import jax
import jax.numpy as jnp
from jax import lax
from jax.experimental import pallas as pl
from jax.experimental.pallas import tpu as pltpu
from jax.experimental.pallas import tpu_sc as plsc

NN = 10000
EE = 160000
DD = 256
GG = 16
CC = 16
HALF = 128

BS = 1000
NBLK = NN // BS

EPAD = 163840
EPT_D = EPAD // 32
NCH_D = EPT_D // 128
EPT_A = EPAD // 16
NCH_A = EPT_A // 128
NPAD = 10240
RPT = NPAD // 16
ZR_D = 10240 // 16


def _onehot(batch_blk):
    return (batch_blk == lax.broadcasted_iota(jnp.int32, (1, GG), 1)).astype(
        jnp.float32)


def _segsum(p, v):
    return lax.dot_general(p, v, (((0,), (0,)), ((), ())),
                           precision=lax.Precision.HIGHEST,
                           preferred_element_type=jnp.float32)


@functools.cache
def _get_sc_deg():
    mesh = plsc.VectorSubcoreMesh(core_axis_name="c", subcore_axis_name="s")
    return pl.kernel(
        _sc_deg_body,
        out_type=jax.ShapeDtypeStruct((2, 10240), jnp.float32),
        mesh=mesh,
        scratch_types=[
            pltpu.VMEM((128,), jnp.int32),
            pltpu.VMEM((128,), jnp.float32),
            pltpu.VMEM_SHARED((10240,), jnp.float32),
        ],
    )


def _sc_deg_body(dst_hbm, w_hbm, z_hbm, deg_hbm, dbuf, wbuf, deg_s):
    c = lax.axis_index("c")
    s = lax.axis_index("s")
    pltpu.sync_copy(z_hbm, deg_s.at[pl.ds(s * ZR_D, ZR_D)])
    plsc.subcore_barrier()
    base = (c * 16 + s) * EPT_D

    def chunk(g, carry):
        off = base + g * 128
        pltpu.sync_copy(dst_hbm.at[pl.ds(off, 128)], dbuf)
        pltpu.sync_copy(w_hbm.at[pl.ds(off, 128)], wbuf)
        pltpu.sync_copy(wbuf, deg_s.at[dbuf], add=True)
        return carry

    lax.fori_loop(0, NCH_D, chunk, 0)
    plsc.subcore_barrier()
    pltpu.sync_copy(deg_s.at[pl.ds(s * ZR_D, ZR_D)],
                    deg_hbm.at[c, pl.ds(s * ZR_D, ZR_D)])


CHPT = NCH_A


@functools.cache
def _get_sc_agg():
    mesh = plsc.VectorSubcoreMesh(core_axis_name="c", subcore_axis_name="s")
    return pl.kernel(
        _sc_agg_body,
        out_type=jax.ShapeDtypeStruct((2, NPAD, HALF), jnp.float32),
        mesh=mesh,
        scratch_types=[
            pltpu.VMEM((128,), jnp.int32),
            pltpu.VMEM((128,), jnp.int32),
            pltpu.VMEM((128,), jnp.int32),
            pltpu.VMEM((128,), jnp.int32),
            pltpu.VMEM((128,), jnp.float32),
            pltpu.VMEM((128,), jnp.float32),
            pltpu.VMEM((128, HALF), jnp.float32),
            pltpu.VMEM((128, HALF), jnp.float32),
            pltpu.VMEM_SHARED((NPAD, HALF), jnp.float32),
            pltpu.SemaphoreType.DMA,
            pltpu.SemaphoreType.DMA,
            pltpu.SemaphoreType.DMA,
            pltpu.SemaphoreType.DMA,
            pltpu.SemaphoreType.DMA,
            pltpu.SemaphoreType.DMA,
            pltpu.SemaphoreType.DMA,
            pltpu.SemaphoreType.DMA,
        ],
    )


def _sc_agg_body(y_hbm, src2_hbm, dst_hbm, w_hbm, zr_hbm, acc_hbm,
                 sbuf0, sbuf1, dbuf0, dbuf1, wbuf0, wbuf1, rows0, rows1,
                 acc_s, sg0, sg1, si0, si1, sd0, sd1, ss0, ss1):
    c = lax.axis_index("c")
    s = lax.axis_index("s")
    sbuf = (sbuf0, sbuf1)
    dbuf = (dbuf0, dbuf1)
    wbuf = (wbuf0, wbuf1)
    rows = (rows0, rows1)
    sg = (sg0, sg1)
    si = (si0, si1)
    sd = (sd0, sd1)
    ss = (ss0, ss1)
    base = s * CHPT

    def src_row(g):
        return src2_hbm.at[c, base + g]

    def dst_row(g):
        return dst_hbm.at[base + g]

    def w_row(g):
        return w_hbm.at[base + g]

    pltpu.sync_copy(src_row(0), sbuf0)
    pltpu.async_copy(y_hbm.at[sbuf0], rows0, sg0)
    pltpu.async_copy(dst_row(0), dbuf0, sd0)
    pltpu.async_copy(w_row(0), wbuf0, sd0)
    pltpu.async_copy(src_row(1), sbuf1, si1)
    pltpu.sync_copy(zr_hbm, acc_s.at[pl.ds(s * RPT, RPT)])
    plsc.subcore_barrier()

    def pair(i, carry):
        g0 = i * 2
        for par in range(2):
            g = g0 + par
            q = 1 - par

            @pl.when(g + 1 < NCH_A)
            def _(g=g, par=par, q=q):
                pltpu.make_async_copy(src_row(g + 1), sbuf[q], si[q]).wait()

                @pl.when(g >= 1)
                def _():
                    pltpu.make_async_copy(rows[q], acc_s.at[dbuf[q]],
                                          ss[q]).wait()

                pltpu.async_copy(y_hbm.at[sbuf[q]], rows[q], sg[q])
                pltpu.async_copy(dst_row(g + 1), dbuf[q], sd[q])
                pltpu.async_copy(w_row(g + 1), wbuf[q], sd[q])

            pltpu.make_async_copy(y_hbm.at[sbuf[par]], rows[par],
                                  sg[par]).wait()

            @pl.when(g + 2 < NCH_A)
            def _(g=g, par=par):
                pltpu.async_copy(src_row(g + 2), sbuf[par], si[par])

            pltpu.make_async_copy(dst_row(g), dbuf[par], sd[par]).wait()
            pltpu.make_async_copy(w_row(g), wbuf[par], sd[par]).wait()

            @plsc.parallel_loop(0, 8, unroll=2)
            def _(m, par=par):
                wv = wbuf[par][pl.ds(m * 16, 16)]
                for l in range(16):
                    wk = wv[l]
                    row = m * 16 + l
                    for j in range(8):
                        sl = pl.ds(j * 16, 16)
                        rows[par][row, sl] = rows[par][row, sl] * wk

            pltpu.async_copy(rows[par], acc_s.at[dbuf[par]], ss[par],
                             add=True)
        return carry

    lax.fori_loop(0, NCH_A // 2, pair, 0)
    for p in range(2):
        pltpu.make_async_copy(rows[p], acc_s.at[dbuf[p]], ss[p]).wait()
    plsc.subcore_barrier()
    pltpu.sync_copy(acc_s.at[pl.ds(s * RPT, RPT)],
                    acc_hbm.at[c, pl.ds(s * RPT, RPT)])


def _kdis_body(degp_ref, dis_ref):
    a = degp_ref[...]
    tot = a[0:1, :] + a[1:2, :] + 1.0
    dis_ref[...] = lax.rsqrt(tot)


def _kdis(degp):
    return pl.pallas_call(
        _kdis_body,
        out_shape=jax.ShapeDtypeStruct((1, 10240), jnp.float32),
    )(degp)


def _k1_body(x_ref, w_ref, dis_ref, y_ref):
    xw = jnp.dot(x_ref[...], w_ref[...], preferred_element_type=jnp.float32)
    y = xw * dis_ref[...]
    y_ref[0] = y[:, :HALF]
    y_ref[1] = y[:, HALF:]


def _k1(x, W, dis2):
    return pl.pallas_call(
        _k1_body,
        grid=(NBLK,),
        in_specs=[pl.BlockSpec((BS, DD), lambda i: (i, 0)),
                  pl.BlockSpec((DD, DD), lambda i: (0, 0)),
                  pl.BlockSpec((BS, 1), lambda i: (i, 0))],
        out_specs=pl.BlockSpec((2, BS, HALF), lambda i: (0, i, 0)),
        out_shape=jax.ShapeDtypeStruct((2, NN, HALF), jnp.float32),
    )(x, W, dis2)


def _k2a_body(acc_ref, y_ref, dis_ref, b_ref, bat_ref, hpre_ref, s1_ref):
    i = pl.program_id(0)
    acc = jnp.concatenate([acc_ref[0], acc_ref[1]], axis=1)
    y = jnp.concatenate([y_ref[0], y_ref[1]], axis=1)
    hpre = dis_ref[...] * (acc + y) + b_ref[...]
    hpre_ref[...] = hpre
    p = _onehot(bat_ref[...])

    @pl.when(i == 0)
    def _():
        s1_ref[...] = jnp.zeros_like(s1_ref)

    s1_ref[...] += _segsum(p, hpre)


def _k2a(acc, y2, dis2, b, batch2):
    return pl.pallas_call(
        _k2a_body,
        grid=(NBLK,),
        in_specs=[pl.BlockSpec((2, BS, HALF), lambda i: (0, i, 0)),
                  pl.BlockSpec((2, BS, HALF), lambda i: (0, i, 0)),
                  pl.BlockSpec((BS, 1), lambda i: (i, 0)),
                  pl.BlockSpec((1, DD), lambda i: (0, 0)),
                  pl.BlockSpec((BS, 1), lambda i: (i, 0))],
        out_specs=[pl.BlockSpec((BS, DD), lambda i: (i, 0)),
                   pl.BlockSpec((GG, DD), lambda i: (0, 0))],
        out_shape=[jax.ShapeDtypeStruct((NN, DD), jnp.float32),
                   jax.ShapeDtypeStruct((GG, DD), jnp.float32)],
    )(acc, y2, dis2, b, batch2)


def _k2b_body(hpre_ref, s1_ref, cnt_ref, ga_ref, bat_ref, out0_ref, s2_ref):
    i = pl.program_id(0)
    mean = s1_ref[...] / cnt_ref[...]
    m2 = mean * ga_ref[...]
    p = _onehot(bat_ref[...])
    out0 = hpre_ref[...] - jnp.dot(p, m2, precision=lax.Precision.HIGHEST,
                                   preferred_element_type=jnp.float32)
    out0_ref[...] = out0

    @pl.when(i == 0)
    def _():
        s2_ref[...] = jnp.zeros_like(s2_ref)

    s2_ref[...] += _segsum(p, out0 * out0)


def _k2b(hpre, S1, cnt, ga, batch2):
    return pl.pallas_call(
        _k2b_body,
        grid=(NBLK,),
        in_specs=[pl.BlockSpec((BS, DD), lambda i: (i, 0)),
                  pl.BlockSpec((GG, DD), lambda i: (0, 0)),
                  pl.BlockSpec((GG, 1), lambda i: (0, 0)),
                  pl.BlockSpec((1, DD), lambda i: (0, 0)),
                  pl.BlockSpec((BS, 1), lambda i: (i, 0))],
        out_specs=[pl.BlockSpec((BS, DD), lambda i: (i, 0)),
                   pl.BlockSpec((GG, DD), lambda i: (0, 0))],
        out_shape=[jax.ShapeDtypeStruct((NN, DD), jnp.float32),
                   jax.ShapeDtypeStruct((GG, DD), jnp.float32)],
    )(hpre, S1, cnt, ga, batch2)


def _k2c_body(out0_ref, s2_ref, cnt_ref, gw_ref, gb_ref, bat_ref, xres_ref,
              h_ref, *, res):
    inv = lax.rsqrt(s2_ref[...] / cnt_ref[...] + 1e-5)
    p = _onehot(bat_ref[...])
    fb = jnp.dot(p, gw_ref[...] * inv, precision=lax.Precision.HIGHEST,
                 preferred_element_type=jnp.float32)
    h = jnp.maximum(out0_ref[...] * fb + gb_ref[...], 0.0)
    if res:
        h = h + xres_ref[...]
    h_ref[...] = h


def _k2c(out0, S2, cnt, gw, gb, batch2, xres, res):
    return pl.pallas_call(
        functools.partial(_k2c_body, res=res),
        grid=(NBLK,),
        in_specs=[pl.BlockSpec((BS, DD), lambda i: (i, 0)),
                  pl.BlockSpec((GG, DD), lambda i: (0, 0)),
                  pl.BlockSpec((GG, 1), lambda i: (0, 0)),
                  pl.BlockSpec((1, DD), lambda i: (0, 0)),
                  pl.BlockSpec((1, DD), lambda i: (0, 0)),
                  pl.BlockSpec((BS, 1), lambda i: (i, 0)),
                  pl.BlockSpec((BS, DD), lambda i: (i, 0))],
        out_specs=pl.BlockSpec((BS, DD), lambda i: (i, 0)),
        out_shape=jax.ShapeDtypeStruct((NN, DD), jnp.float32),
    )(out0, S2, cnt, gw, gb, batch2, xres)


def _kcnt_body(bat_ref, cnt_ref):
    i = pl.program_id(0)
    p = _onehot(bat_ref[...])

    @pl.when(i == 0)
    def _():
        cnt_ref[...] = jnp.zeros_like(cnt_ref)

    cnt_ref[...] += _segsum(p, jnp.ones((BS, HALF), jnp.float32))


def _kcnt(batch2):
    return pl.pallas_call(
        _kcnt_body,
        grid=(NBLK,),
        in_specs=[pl.BlockSpec((BS, 1), lambda i: (i, 0))],
        out_specs=pl.BlockSpec((GG, HALF), lambda i: (0, 0)),
        out_shape=jax.ShapeDtypeStruct((GG, HALF), jnp.float32),
    )(batch2)


def _k3a_body(h_ref, bat_ref, s_ref):
    i = pl.program_id(0)
    p = _onehot(bat_ref[...])

    @pl.when(i == 0)
    def _():
        s_ref[...] = jnp.zeros_like(s_ref)

    s_ref[...] += _segsum(p, h_ref[...])


def _k3a(h, batch2):
    return pl.pallas_call(
        _k3a_body,
        grid=(NBLK,),
        in_specs=[pl.BlockSpec((BS, DD), lambda i: (i, 0)),
                  pl.BlockSpec((BS, 1), lambda i: (i, 0))],
        out_specs=pl.BlockSpec((GG, DD), lambda i: (0, 0)),
        out_shape=jax.ShapeDtypeStruct((GG, DD), jnp.float32),
    )(h, batch2)


def _k3b_body(pool_ref, cnt_ref, wdin_ref, bdin_ref, wd1_ref, bd1_ref,
              wdout_ref, bdout_ref, o_ref):
    m = pool_ref[...] / cnt_ref[...]
    z = jnp.maximum(
        jnp.dot(m, wdin_ref[...], preferred_element_type=jnp.float32)
        + bdin_ref[...], 0.0)
    z = jnp.maximum(
        jnp.dot(z, wd1_ref[...], preferred_element_type=jnp.float32)
        + bd1_ref[...], 0.0)
    o_ref[...] = (jnp.dot(z, wdout_ref[...], preferred_element_type=jnp.float32)
                  + bdout_ref[...])


def _k3b(pool, cnt, Wdin, bdin, Wd1, bd1, Wdout, bdout):
    return pl.pallas_call(
        _k3b_body,
        out_shape=jax.ShapeDtypeStruct((GG, CC), jnp.float32),
    )(pool, cnt, Wdin, bdin, Wd1, bd1, Wdout, bdout)


def kernel(inputs, edge_index, batch, edge_weight, W0, b0, gnw0, gnb0, gna0,
           W1, b1, gnw1, gnb1, gna1, W2, b2, gnw2, gnb2, gna2,
           Wdin, bdin, Wd1, bd1, Wdout, bdout):
    pad = EPAD - EE
    srcp = jnp.concatenate([edge_index[0], jnp.zeros((pad,), jnp.int32)])
    dstp = jnp.concatenate([edge_index[1], jnp.zeros((pad,), jnp.int32)])
    wp = jnp.concatenate([edge_weight, jnp.zeros((pad,), jnp.float32)])
    zdeg = jnp.zeros((ZR_D,), jnp.float32)
    zrows = jnp.zeros((RPT, HALF), jnp.float32)
    batch2 = batch.reshape(NN, 1)

    src2 = jnp.stack([srcp, srcp + NN]).reshape(2, EPAD // 128, 128)
    dstr = dstp.reshape(EPAD // 128, 128)
    wr = wp.reshape(EPAD // 128, 128)

    degp = _get_sc_deg()(dstp, wp, zdeg)
    dis2 = _kdis(degp).reshape(10240, 1)[:NN]
    cnt = _kcnt(batch2)[:, :1]

    h = inputs
    layers = [(W0, b0, gnw0, gnb0, gna0, False),
              (W1, b1, gnw1, gnb1, gna1, True),
              (W2, b2, gnw2, gnb2, gna2, True)]
    for (W, b, gw, gb, ga, res) in layers:
        y2 = _k1(h, W, dis2)
        acc = _get_sc_agg()(y2.reshape(2 * NN, HALF), src2, dstr, wr, zrows)
        hpre, S1 = _k2a(acc, y2, dis2, b.reshape(1, DD), batch2)
        out0, S2 = _k2b(hpre, S1, cnt, ga.reshape(1, DD), batch2)
        h = _k2c(out0, S2, cnt, gw.reshape(1, DD), gb.reshape(1, DD),
                 batch2, h, res)

    pool = _k3a(h, batch2)
    return _k3b(pool, cnt, Wdin, bdin.reshape(1, DD), Wd1, bd1.reshape(1, DD),
                Wdout, bdout.reshape(1, CC))


# speedup vs baseline: 7.9464x; 1.0011x over previous
import functools

import jax
import jax.numpy as jnp
from jax import lax
from jax.experimental import pallas as pl
from jax.experimental.pallas import tpu as pltpu
from jax.experimental.pallas import tpu_sc as plsc

NN = 10000
EE = 160000
DD = 256
GG = 16
CC = 16
HALF = 128

BS = 1000
NBLK = NN // BS

EPAD = 163840
EPT_D = EPAD // 32
NCH_D = EPT_D // 128
EPT_A = EPAD // 16
NCH_A = EPT_A // 128
NPAD = 10240
RPT = NPAD // 16
ZR_D = 10240 // 16


def _onehot(batch_blk):
    return (batch_blk == lax.broadcasted_iota(jnp.int32, (1, GG), 1)).astype(
        jnp.float32)


def _segsum(p, v):
    return lax.dot_general(p, v, (((0,), (0,)), ((), ())),
                           precision=lax.Precision.HIGHEST,
                           preferred_element_type=jnp.float32)


@functools.cache
def _get_sc_deg():
    mesh = plsc.VectorSubcoreMesh(core_axis_name="c", subcore_axis_name="s")
    return pl.kernel(
        _sc_deg_body,
        out_type=jax.ShapeDtypeStruct((2, 10240), jnp.float32),
        mesh=mesh,
        scratch_types=[
            pltpu.VMEM((128,), jnp.int32),
            pltpu.VMEM((128,), jnp.float32),
            pltpu.VMEM_SHARED((10240,), jnp.float32),
        ],
    )


def _sc_deg_body(dst_hbm, w_hbm, z_hbm, deg_hbm, dbuf, wbuf, deg_s):
    c = lax.axis_index("c")
    s = lax.axis_index("s")
    pltpu.sync_copy(z_hbm, deg_s.at[pl.ds(s * ZR_D, ZR_D)])
    plsc.subcore_barrier()
    base = (c * 16 + s) * EPT_D

    def chunk(g, carry):
        off = base + g * 128
        pltpu.sync_copy(dst_hbm.at[pl.ds(off, 128)], dbuf)
        pltpu.sync_copy(w_hbm.at[pl.ds(off, 128)], wbuf)
        pltpu.sync_copy(wbuf, deg_s.at[dbuf], add=True)
        return carry

    lax.fori_loop(0, NCH_D, chunk, 0)
    plsc.subcore_barrier()
    pltpu.sync_copy(deg_s.at[pl.ds(s * ZR_D, ZR_D)],
                    deg_hbm.at[c, pl.ds(s * ZR_D, ZR_D)])


CHPT = NCH_A


@functools.cache
def _get_sc_agg():
    mesh = plsc.VectorSubcoreMesh(core_axis_name="c", subcore_axis_name="s")
    return pl.kernel(
        _sc_agg_body,
        out_type=jax.ShapeDtypeStruct((2, NPAD, HALF), jnp.float32),
        mesh=mesh,
        scratch_types=[
            pltpu.VMEM((128,), jnp.int32),
            pltpu.VMEM((128,), jnp.int32),
            pltpu.VMEM((128,), jnp.int32),
            pltpu.VMEM((128,), jnp.int32),
            pltpu.VMEM((128,), jnp.float32),
            pltpu.VMEM((128,), jnp.float32),
            pltpu.VMEM((128, HALF), jnp.float32),
            pltpu.VMEM((128, HALF), jnp.float32),
            pltpu.VMEM_SHARED((NPAD, HALF), jnp.float32),
            pltpu.SemaphoreType.DMA,
            pltpu.SemaphoreType.DMA,
            pltpu.SemaphoreType.DMA,
            pltpu.SemaphoreType.DMA,
            pltpu.SemaphoreType.DMA,
            pltpu.SemaphoreType.DMA,
            pltpu.SemaphoreType.DMA,
            pltpu.SemaphoreType.DMA,
        ],
    )


def _sc_agg_body(y_hbm, src2_hbm, dst_hbm, w_hbm, zr_hbm, acc_hbm,
                 sbuf0, sbuf1, dbuf0, dbuf1, wbuf0, wbuf1, rows0, rows1,
                 acc_s, sga0, sga1, sgb0, sgb1, si0, si1, sd0, sd1):
    c = lax.axis_index("c")
    s = lax.axis_index("s")
    sbuf = (sbuf0, sbuf1)
    dbuf = (dbuf0, dbuf1)
    wbuf = (wbuf0, wbuf1)
    rows = (rows0, rows1)
    sga = (sga0, sga1)
    sgb = (sgb0, sgb1)
    si = (si0, si1)
    sd = (sd0, sd1)
    base = s * CHPT

    def src_row(g):
        return src2_hbm.at[c, base + g]

    def dst_row(g):
        return dst_hbm.at[base + g]

    def w_row(g):
        return w_hbm.at[base + g]

    def start_gather(b):
        pltpu.async_copy(y_hbm.at[sbuf[b].at[pl.ds(0, 64)]],
                         rows[b].at[pl.ds(0, 64)], sga[b])
        pltpu.async_copy(y_hbm.at[sbuf[b].at[pl.ds(64, 64)]],
                         rows[b].at[pl.ds(64, 64)], sgb[b])

    def wait_gather(b):
        pltpu.make_async_copy(y_hbm.at[sbuf[b].at[pl.ds(0, 64)]],
                              rows[b].at[pl.ds(0, 64)], sga[b]).wait()
        pltpu.make_async_copy(y_hbm.at[sbuf[b].at[pl.ds(64, 64)]],
                              rows[b].at[pl.ds(64, 64)], sgb[b]).wait()

    pltpu.sync_copy(src_row(0), sbuf0)
    start_gather(0)
    pltpu.async_copy(dst_row(0), dbuf0, sd0)
    pltpu.async_copy(w_row(0), wbuf0, sd0)
    pltpu.async_copy(src_row(1), sbuf1, si1)
    pltpu.async_copy(dst_row(1), dbuf1, sd1)
    pltpu.async_copy(w_row(1), wbuf1, sd1)
    pltpu.sync_copy(zr_hbm, acc_s.at[pl.ds(s * RPT, RPT)])
    plsc.subcore_barrier()

    def pair(i, carry):
        g0 = i * 2
        for par in range(2):
            g = g0 + par
            q = 1 - par

            @pl.when(g + 1 < NCH_A)
            def _(g=g, q=q):
                pltpu.make_async_copy(src_row(g + 1), sbuf[q], si[q]).wait()
                start_gather(q)

            wait_gather(par)

            @pl.when(g + 2 < NCH_A)
            def _(g=g, par=par):
                pltpu.async_copy(src_row(g + 2), sbuf[par], si[par])

            pltpu.make_async_copy(dst_row(g), dbuf[par], sd[par]).wait()
            pltpu.make_async_copy(w_row(g), wbuf[par], sd[par]).wait()

            def mul(m, carry2, par=par):
                wv = wbuf[par][pl.ds(m * 16, 16)]
                for l in range(16):
                    wk = wv[l]
                    row = m * 16 + l
                    for j in range(8):
                        sl = pl.ds(j * 16, 16)
                        rows[par][row, sl] = rows[par][row, sl] * wk
                return carry2

            lax.fori_loop(0, 8, mul, 0)
            pltpu.sync_copy(rows[par], acc_s.at[dbuf[par]], add=True)

            @pl.when(g + 2 < NCH_A)
            def _(g=g, par=par):
                pltpu.async_copy(dst_row(g + 2), dbuf[par], sd[par])
                pltpu.async_copy(w_row(g + 2), wbuf[par], sd[par])
        return carry

    lax.fori_loop(0, NCH_A // 2, pair, 0)
    plsc.subcore_barrier()
    pltpu.sync_copy(acc_s.at[pl.ds(s * RPT, RPT)],
                    acc_hbm.at[c, pl.ds(s * RPT, RPT)])


def _kdis_body(degp_ref, dis_ref):
    a = degp_ref[...]
    tot = a[0:1, :] + a[1:2, :] + 1.0
    dis_ref[...] = lax.rsqrt(tot)


def _kdis(degp):
    return pl.pallas_call(
        _kdis_body,
        out_shape=jax.ShapeDtypeStruct((1, 10240), jnp.float32),
    )(degp)


def _k1_body(x_ref, w_ref, dis_ref, y_ref):
    xw = jnp.dot(x_ref[...], w_ref[...], preferred_element_type=jnp.float32)
    y = xw * dis_ref[...]
    y_ref[0] = y[:, :HALF]
    y_ref[1] = y[:, HALF:]


def _k1(x, W, dis2):
    return pl.pallas_call(
        _k1_body,
        grid=(NBLK,),
        in_specs=[pl.BlockSpec((BS, DD), lambda i: (i, 0)),
                  pl.BlockSpec((DD, DD), lambda i: (0, 0)),
                  pl.BlockSpec((BS, 1), lambda i: (i, 0))],
        out_specs=pl.BlockSpec((2, BS, HALF), lambda i: (0, i, 0)),
        out_shape=jax.ShapeDtypeStruct((2, NN, HALF), jnp.float32),
    )(x, W, dis2)


def _k2a_body(acc_ref, y_ref, dis_ref, b_ref, bat_ref, hpre_ref, s1_ref):
    i = pl.program_id(0)
    acc = jnp.concatenate([acc_ref[0], acc_ref[1]], axis=1)
    y = jnp.concatenate([y_ref[0], y_ref[1]], axis=1)
    hpre = dis_ref[...] * (acc + y) + b_ref[...]
    hpre_ref[...] = hpre
    p = _onehot(bat_ref[...])

    @pl.when(i == 0)
    def _():
        s1_ref[...] = jnp.zeros_like(s1_ref)

    s1_ref[...] += _segsum(p, hpre)


def _k2a(acc, y2, dis2, b, batch2):
    return pl.pallas_call(
        _k2a_body,
        grid=(NBLK,),
        in_specs=[pl.BlockSpec((2, BS, HALF), lambda i: (0, i, 0)),
                  pl.BlockSpec((2, BS, HALF), lambda i: (0, i, 0)),
                  pl.BlockSpec((BS, 1), lambda i: (i, 0)),
                  pl.BlockSpec((1, DD), lambda i: (0, 0)),
                  pl.BlockSpec((BS, 1), lambda i: (i, 0))],
        out_specs=[pl.BlockSpec((BS, DD), lambda i: (i, 0)),
                   pl.BlockSpec((GG, DD), lambda i: (0, 0))],
        out_shape=[jax.ShapeDtypeStruct((NN, DD), jnp.float32),
                   jax.ShapeDtypeStruct((GG, DD), jnp.float32)],
    )(acc, y2, dis2, b, batch2)


def _k2b_body(hpre_ref, s1_ref, cnt_ref, ga_ref, bat_ref, out0_ref, s2_ref):
    i = pl.program_id(0)
    mean = s1_ref[...] / cnt_ref[...]
    m2 = mean * ga_ref[...]
    p = _onehot(bat_ref[...])
    out0 = hpre_ref[...] - jnp.dot(p, m2, precision=lax.Precision.HIGHEST,
                                   preferred_element_type=jnp.float32)
    out0_ref[...] = out0

    @pl.when(i == 0)
    def _():
        s2_ref[...] = jnp.zeros_like(s2_ref)

    s2_ref[...] += _segsum(p, out0 * out0)


def _k2b(hpre, S1, cnt, ga, batch2):
    return pl.pallas_call(
        _k2b_body,
        grid=(NBLK,),
        in_specs=[pl.BlockSpec((BS, DD), lambda i: (i, 0)),
                  pl.BlockSpec((GG, DD), lambda i: (0, 0)),
                  pl.BlockSpec((GG, 1), lambda i: (0, 0)),
                  pl.BlockSpec((1, DD), lambda i: (0, 0)),
                  pl.BlockSpec((BS, 1), lambda i: (i, 0))],
        out_specs=[pl.BlockSpec((BS, DD), lambda i: (i, 0)),
                   pl.BlockSpec((GG, DD), lambda i: (0, 0))],
        out_shape=[jax.ShapeDtypeStruct((NN, DD), jnp.float32),
                   jax.ShapeDtypeStruct((GG, DD), jnp.float32)],
    )(hpre, S1, cnt, ga, batch2)


def _k2c_body(out0_ref, s2_ref, cnt_ref, gw_ref, gb_ref, bat_ref, xres_ref,
              h_ref, *, res):
    inv = lax.rsqrt(s2_ref[...] / cnt_ref[...] + 1e-5)
    p = _onehot(bat_ref[...])
    fb = jnp.dot(p, gw_ref[...] * inv, precision=lax.Precision.HIGHEST,
                 preferred_element_type=jnp.float32)
    h = jnp.maximum(out0_ref[...] * fb + gb_ref[...], 0.0)
    if res:
        h = h + xres_ref[...]
    h_ref[...] = h


def _k2c(out0, S2, cnt, gw, gb, batch2, xres, res):
    return pl.pallas_call(
        functools.partial(_k2c_body, res=res),
        grid=(NBLK,),
        in_specs=[pl.BlockSpec((BS, DD), lambda i: (i, 0)),
                  pl.BlockSpec((GG, DD), lambda i: (0, 0)),
                  pl.BlockSpec((GG, 1), lambda i: (0, 0)),
                  pl.BlockSpec((1, DD), lambda i: (0, 0)),
                  pl.BlockSpec((1, DD), lambda i: (0, 0)),
                  pl.BlockSpec((BS, 1), lambda i: (i, 0)),
                  pl.BlockSpec((BS, DD), lambda i: (i, 0))],
        out_specs=pl.BlockSpec((BS, DD), lambda i: (i, 0)),
        out_shape=jax.ShapeDtypeStruct((NN, DD), jnp.float32),
    )(out0, S2, cnt, gw, gb, batch2, xres)


def _kcnt_body(bat_ref, cnt_ref):
    i = pl.program_id(0)
    p = _onehot(bat_ref[...])

    @pl.when(i == 0)
    def _():
        cnt_ref[...] = jnp.zeros_like(cnt_ref)

    cnt_ref[...] += _segsum(p, jnp.ones((BS, HALF), jnp.float32))


def _kcnt(batch2):
    return pl.pallas_call(
        _kcnt_body,
        grid=(NBLK,),
        in_specs=[pl.BlockSpec((BS, 1), lambda i: (i, 0))],
        out_specs=pl.BlockSpec((GG, HALF), lambda i: (0, 0)),
        out_shape=jax.ShapeDtypeStruct((GG, HALF), jnp.float32),
    )(batch2)


def _k3a_body(h_ref, bat_ref, s_ref):
    i = pl.program_id(0)
    p = _onehot(bat_ref[...])

    @pl.when(i == 0)
    def _():
        s_ref[...] = jnp.zeros_like(s_ref)

    s_ref[...] += _segsum(p, h_ref[...])


def _k3a(h, batch2):
    return pl.pallas_call(
        _k3a_body,
        grid=(NBLK,),
        in_specs=[pl.BlockSpec((BS, DD), lambda i: (i, 0)),
                  pl.BlockSpec((BS, 1), lambda i: (i, 0))],
        out_specs=pl.BlockSpec((GG, DD), lambda i: (0, 0)),
        out_shape=jax.ShapeDtypeStruct((GG, DD), jnp.float32),
    )(h, batch2)


def _k3b_body(pool_ref, cnt_ref, wdin_ref, bdin_ref, wd1_ref, bd1_ref,
              wdout_ref, bdout_ref, o_ref):
    m = pool_ref[...] / cnt_ref[...]
    z = jnp.maximum(
        jnp.dot(m, wdin_ref[...], preferred_element_type=jnp.float32)
        + bdin_ref[...], 0.0)
    z = jnp.maximum(
        jnp.dot(z, wd1_ref[...], preferred_element_type=jnp.float32)
        + bd1_ref[...], 0.0)
    o_ref[...] = (jnp.dot(z, wdout_ref[...], preferred_element_type=jnp.float32)
                  + bdout_ref[...])


def _k3b(pool, cnt, Wdin, bdin, Wd1, bd1, Wdout, bdout):
    return pl.pallas_call(
        _k3b_body,
        out_shape=jax.ShapeDtypeStruct((GG, CC), jnp.float32),
    )(pool, cnt, Wdin, bdin, Wd1, bd1, Wdout, bdout)


def kernel(inputs, edge_index, batch, edge_weight, W0, b0, gnw0, gnb0, gna0,
           W1, b1, gnw1, gnb1, gna1, W2, b2, gnw2, gnb2, gna2,
           Wdin, bdin, Wd1, bd1, Wdout, bdout):
    pad = EPAD - EE
    srcp = jnp.concatenate([edge_index[0], jnp.zeros((pad,), jnp.int32)])
    dstp = jnp.concatenate([edge_index[1], jnp.zeros((pad,), jnp.int32)])
    wp = jnp.concatenate([edge_weight, jnp.zeros((pad,), jnp.float32)])
    zdeg = jnp.zeros((ZR_D,), jnp.float32)
    zrows = jnp.zeros((RPT, HALF), jnp.float32)
    batch2 = batch.reshape(NN, 1)

    src2 = jnp.stack([srcp, srcp + NN]).reshape(2, EPAD // 128, 128)
    dstr = dstp.reshape(EPAD // 128, 128)
    wr = wp.reshape(EPAD // 128, 128)

    degp = _get_sc_deg()(dstp, wp, zdeg)
    dis2 = _kdis(degp).reshape(10240, 1)[:NN]
    cnt = _kcnt(batch2)[:, :1]

    h = inputs
    layers = [(W0, b0, gnw0, gnb0, gna0, False),
              (W1, b1, gnw1, gnb1, gna1, True),
              (W2, b2, gnw2, gnb2, gna2, True)]
    for (W, b, gw, gb, ga, res) in layers:
        y2 = _k1(h, W, dis2)
        acc = _get_sc_agg()(y2.reshape(2 * NN, HALF), src2, dstr, wr, zrows)
        hpre, S1 = _k2a(acc, y2, dis2, b.reshape(1, DD), batch2)
        out0, S2 = _k2b(hpre, S1, cnt, ga.reshape(1, DD), batch2)
        h = _k2c(out0, S2, cnt, gw.reshape(1, DD), gb.reshape(1, DD),
                 batch2, h, res)

    pool = _k3a(h, batch2)
    return _k3b(pool, cnt, Wdin, bdin.reshape(1, DD), Wd1, bd1.reshape(1, DD),
                Wdout, bdout.reshape(1, CC))
